# baseline (speedup 1.0000x reference)
import functools

import jax
import jax.numpy as jnp
from jax import lax
from jax.experimental import pallas as pl
from jax.experimental.pallas import tpu as pltpu
from jax.experimental.pallas import tpu_sc as plsc

B = 256
N = 2048
K = 204
L = 16
NCHUNK = N // L
UNROLL = 8
CUNROLL = 16


def _build_sc_kernel():
    info = plsc.get_sparse_core_info()
    nc, ns = info.num_cores, info.num_subcores
    nw = nc * ns
    rows_per_w = B // nw
    mesh = plsc.VectorSubcoreMesh(core_axis_name="c", subcore_axis_name="s")

    @functools.partial(
        pl.kernel,
        mesh=mesh,
        out_type=jax.ShapeDtypeStruct((4, nw, L), jnp.float32),
        compiler_params=pltpu.CompilerParams(needs_layout_passes=False),
        scratch_types=[
            pltpu.VMEM((L,), jnp.float32),
            pltpu.VMEM((L,), jnp.float32),
            pltpu.VMEM((2 * N,), jnp.float32),
            pltpu.VMEM((2 * N,), jnp.float32),
            pltpu.VMEM((2 * N,), jnp.float32),
            pltpu.VMEM((2 * N,), jnp.float32),
            pltpu.VMEM((2 * N,), jnp.float32),
            pltpu.VMEM((2 * N,), jnp.float32),
            pltpu.VMEM((N,), jnp.uint32),
            pltpu.VMEM((N,), jnp.float32),
            pltpu.VMEM((L,), jnp.float32),
            pltpu.SemaphoreType.DMA,
            pltpu.SemaphoreType.DMA,
        ],
    )
    def sc_loss(h_hbm, p1_hbm, dx_hbm, p2_hbm,
                out_hbm, hbuf0, hbuf1, p1b0, p1b1, dxb0, dxb1, p2b0, p2b1,
                dbuf, tbuf, obuf, sem0, sem1):
        wid = lax.axis_index("s") * nc + lax.axis_index("c")
        zf = jnp.zeros((L,), jnp.float32)
        zi = jnp.zeros((L,), jnp.int32)
        kvec = jnp.full((L,), K, jnp.int32)
        one_u = jnp.ones((L,), jnp.uint32)
        iota16 = lax.iota(jnp.int32, L)
        idx0 = iota16 * 2
        sems = (sem0, sem1)
        srcs = (h_hbm, p1_hbm, dx_hbm, p2_hbm)
        slots = ((hbuf0, p1b0, dxb0, p2b0), (hbuf1, p1b1, dxb1, p2b1))

        def issue(r, s):
            b = wid * rows_per_w + r
            return [pltpu.async_copy(src.at[b], dst, sems[s])
                    for src, dst in zip(srcs, slots[s])]

        def do_row(s, accs):
            acc_s, acc_r, acc_t, acc_rem = accs
            hbuf, p1b, dxb, p2b = slots[s]
            h = [plsc.load_gather(hbuf, [jnp.full((L,), j, jnp.int32)])
                 for j in range(8)]

            @plsc.parallel_loop(0, NCHUNK, 1, unroll=UNROLL,
                                carry=(acc_s, acc_r, acc_t))
            def comp(cc, c):
                a_s, a_r, a_t = c
                sl = pl.ds(cc * L, L)
                ix = idx0 + cc * (2 * L)
                iy = ix + 1
                px = (plsc.load_gather(p1b, [ix])
                      + plsc.load_gather(dxb, [ix]))
                py = (plsc.load_gather(p1b, [iy])
                      + plsc.load_gather(dxb, [iy]))
                a_r = (a_r + jnp.maximum(jnp.abs(px) - 1.0, 0.0)
                       + jnp.maximum(jnp.abs(py) - 1.0, 0.0))
                inv = 1.0 / (h[6] * px + h[7] * py + 1.0)
                ex = ((h[0] * px + h[1] * py + h[2]) * inv
                      - plsc.load_gather(p2b, [ix]))
                ey = ((h[3] * px + h[4] * py + h[5]) * inv
                      - plsc.load_gather(p2b, [iy]))
                d2 = ex * ex + ey * ey
                w = jnp.exp(-d2)
                t2 = w * w * d2
                a_s = a_s + w
                a_t = a_t + t2
                dbuf[sl] = plsc.bitcast(d2, jnp.uint32)
                tbuf[sl] = t2
                return (a_s, a_r, a_t)

            acc_s, acc_r, acc_t = comp

            def bit_body(j, p):
                sh = jnp.full((L,), 30, jnp.int32) - jnp.full((L,), j, jnp.int32)
                cand = p | (one_u << sh.astype(jnp.uint32))

                def cnt_body(i, c):
                    for u in range(CUNROLL):
                        v = dbuf[pl.ds((i * CUNROLL + u) * L, L)]
                        c = c + plsc.all_reduce_population_count(v >= cand)
                    return c

                cnt = lax.fori_loop(0, NCHUNK // CUNROLL, cnt_body, zi)
                return jnp.where(cnt >= kvec, cand, p)

            t = lax.fori_loop(0, 31, bit_body, jnp.zeros((L,), jnp.uint32))

            tp1 = t + one_u

            def fin(i, c):
                cg, sg = c
                for u in range(UNROLL):
                    sl = pl.ds((i * UNROLL + u) * L, L)
                    m = dbuf[sl] >= tp1
                    cg = cg + plsc.all_reduce_population_count(m)
                    sg = sg + jnp.where(m, tbuf[sl], zf)
                return (cg, sg)

            cgt, sgt = lax.fori_loop(0, NCHUNK // UNROLL, fin, (zi, zf))
            dt = plsc.bitcast(t, jnp.float32)
            gval = jnp.exp(-2.0 * dt) * dt
            tie = (kvec - cgt).astype(jnp.float32) * gval * (1.0 / L)
            acc_rem = acc_rem + sgt + tie
            return (acc_s, acc_r, acc_t, acc_rem)

        accs = (zf, zf, zf, zf)
        pending = issue(0, 0)
        for r in range(rows_per_w):
            for cp in pending:
                cp.wait()
            s = r % 2
            if r + 1 < rows_per_w:
                pending = issue(r + 1, 1 - s)
            accs = do_row(s, accs)

        acc_s, acc_r, acc_t, acc_rem = accs
        obuf[...] = acc_s
        pltpu.sync_copy(obuf, out_hbm.at[0, wid])
        obuf[...] = acc_r
        pltpu.sync_copy(obuf, out_hbm.at[1, wid])
        obuf[...] = acc_t
        pltpu.sync_copy(obuf, out_hbm.at[2, wid])
        obuf[...] = acc_rem
        pltpu.sync_copy(obuf, out_hbm.at[3, wid])

    return sc_loss


_SC_LOSS = _build_sc_kernel()


def kernel(H_out, dx, pts1, pts2):
    h_pad = jnp.pad(H_out, ((0, 0), (0, L - 8)))
    parts = _SC_LOSS(h_pad,
                     pts1.reshape(B, 2 * N),
                     dx.reshape(B, 2 * N),
                     pts2.reshape(B, 2 * N))
    s = jnp.sum(parts[0])
    r = jnp.sum(parts[1])
    total = jnp.sum(parts[2])
    rem = jnp.sum(parts[3])
    mse = (total - rem) / (s * s) / (B * N * 2)
    return mse + r / N


# speedup vs baseline: 1.2029x; 1.2029x over previous
import functools

import jax
import jax.numpy as jnp
from jax import lax
from jax.experimental import pallas as pl
from jax.experimental.pallas import tpu as pltpu
from jax.experimental.pallas import tpu_sc as plsc

B = 256
N = 2048
K = 204
L = 16
NCHUNK = N // L
UNROLL = 8
CUNROLL = 16


def _build_sc_kernel():
    info = plsc.get_sparse_core_info()
    nc, ns = info.num_cores, info.num_subcores
    nw = nc * ns
    rows_per_w = B // nw
    mesh = plsc.VectorSubcoreMesh(core_axis_name="c", subcore_axis_name="s")

    @functools.partial(
        pl.kernel,
        mesh=mesh,
        out_type=jax.ShapeDtypeStruct((4, nw, L), jnp.float32),
        compiler_params=pltpu.CompilerParams(needs_layout_passes=False),
        scratch_types=[
            pltpu.VMEM((2, 8, L), jnp.float32),
            pltpu.VMEM((2, N), jnp.float32),
            pltpu.VMEM((2, N), jnp.float32),
            pltpu.VMEM((2, N), jnp.float32),
            pltpu.VMEM((2, N), jnp.float32),
            pltpu.VMEM((2, N), jnp.float32),
            pltpu.VMEM((2, N), jnp.float32),
            pltpu.VMEM((N,), jnp.uint32),
            pltpu.VMEM((N,), jnp.float32),
            pltpu.VMEM((L,), jnp.float32),
            pltpu.SemaphoreType.DMA,
            pltpu.SemaphoreType.DMA,
        ],
    )
    def sc_loss(h_hbm, p1x_hbm, p1y_hbm, dxx_hbm, dxy_hbm, p2x_hbm, p2y_hbm,
                out_hbm, hbuf, p1xb, p1yb, dxxb, dxyb, p2xb, p2yb,
                dbuf, tbuf, obuf, sem0, sem1):
        wid = lax.axis_index("s") * nc + lax.axis_index("c")
        zf = jnp.zeros((L,), jnp.float32)
        zi = jnp.zeros((L,), jnp.int32)
        kvec = jnp.full((L,), K, jnp.int32)
        one_u = jnp.ones((L,), jnp.uint32)
        sems = (sem0, sem1)
        srcs = (h_hbm, p1x_hbm, p1y_hbm, dxx_hbm, dxy_hbm, p2x_hbm, p2y_hbm)
        dsts = (hbuf, p1xb, p1yb, dxxb, dxyb, p2xb, p2yb)

        def issue(r, s):
            b = wid * rows_per_w + r
            return [pltpu.async_copy(src.at[b], dst.at[s], sems[s])
                    for src, dst in zip(srcs, dsts)]

        def do_row(s, accs):
            acc_s, acc_r, acc_t, acc_rem = accs
            h = [hbuf[s, j] for j in range(8)]

            @plsc.parallel_loop(0, NCHUNK, 1, unroll=UNROLL,
                                carry=(acc_s, acc_r, acc_t))
            def comp(cc, c):
                a_s, a_r, a_t = c
                sl = pl.ds(cc * L, L)
                px = p1xb[s, sl] + dxxb[s, sl]
                py = p1yb[s, sl] + dxyb[s, sl]
                a_r = (a_r + jnp.maximum(jnp.abs(px) - 1.0, 0.0)
                       + jnp.maximum(jnp.abs(py) - 1.0, 0.0))
                inv = 1.0 / (h[6] * px + h[7] * py + 1.0)
                ex = (h[0] * px + h[1] * py + h[2]) * inv - p2xb[s, sl]
                ey = (h[3] * px + h[4] * py + h[5]) * inv - p2yb[s, sl]
                d2 = ex * ex + ey * ey
                w = jnp.exp(-d2)
                t2 = w * w * d2
                a_s = a_s + w
                a_t = a_t + t2
                dbuf[sl] = plsc.bitcast(d2, jnp.uint32)
                tbuf[sl] = t2
                return (a_s, a_r, a_t)

            acc_s, acc_r, acc_t = comp

            def bit_body(j, p):
                sh = jnp.full((L,), 30, jnp.int32) - jnp.full((L,), j, jnp.int32)
                cand = p | (one_u << sh.astype(jnp.uint32))

                def cnt_body(i, c):
                    for u in range(CUNROLL):
                        v = dbuf[pl.ds((i * CUNROLL + u) * L, L)]
                        c = c + plsc.all_reduce_population_count(v >= cand)
                    return c

                cnt = lax.fori_loop(0, NCHUNK // CUNROLL, cnt_body, zi)
                return jnp.where(cnt >= kvec, cand, p)

            t = lax.fori_loop(0, 31, bit_body, jnp.zeros((L,), jnp.uint32))

            tp1 = t + one_u

            @plsc.parallel_loop(0, NCHUNK, 1, unroll=UNROLL, carry=(zi, zf))
            def fin(cc, c):
                cg, sg = c
                sl = pl.ds(cc * L, L)
                m = dbuf[sl] >= tp1
                cg = cg + plsc.all_reduce_population_count(m)
                sg = sg + jnp.where(m, tbuf[sl], zf)
                return (cg, sg)

            cgt, sgt = fin
            dt = plsc.bitcast(t, jnp.float32)
            gval = jnp.exp(-2.0 * dt) * dt
            tie = (kvec - cgt).astype(jnp.float32) * gval * (1.0 / L)
            acc_rem = acc_rem + sgt + tie
            return (acc_s, acc_r, acc_t, acc_rem)

        accs = (zf, zf, zf, zf)
        pending = issue(0, 0)
        for r in range(rows_per_w):
            for cp in pending:
                cp.wait()
            s = r % 2
            if r + 1 < rows_per_w:
                pending = issue(r + 1, 1 - s)
            accs = do_row(s, accs)

        acc_s, acc_r, acc_t, acc_rem = accs
        obuf[...] = acc_s
        pltpu.sync_copy(obuf, out_hbm.at[0, wid])
        obuf[...] = acc_r
        pltpu.sync_copy(obuf, out_hbm.at[1, wid])
        obuf[...] = acc_t
        pltpu.sync_copy(obuf, out_hbm.at[2, wid])
        obuf[...] = acc_rem
        pltpu.sync_copy(obuf, out_hbm.at[3, wid])

    return sc_loss


_SC_LOSS = _build_sc_kernel()


def kernel(H_out, dx, pts1, pts2):
    h_b = jnp.broadcast_to(H_out[:, :, None], (B, 8, L))
    parts = _SC_LOSS(h_b,
                     pts1[:, :, 0], pts1[:, :, 1],
                     dx[:, :, 0], dx[:, :, 1],
                     pts2[:, :, 0], pts2[:, :, 1])
    s = jnp.sum(parts[0])
    r = jnp.sum(parts[1])
    total = jnp.sum(parts[2])
    rem = jnp.sum(parts[3])
    mse = (total - rem) / (s * s) / (B * N * 2)
    return mse + r / N
